# baseline (speedup 1.0000x reference)
import functools

import jax
import jax.numpy as jnp
from jax import lax
from jax.experimental import pallas as pl
from jax.experimental.pallas import tpu as pltpu
from jax.experimental.pallas import tpu_sc as plsc

NC = 2
NS = 16
NW = NC * NS
LANES = 16
CHUNK = 8


def _make_sc_add(B, S, H):
    pos_per_w = S // NW
    n_chunks = pos_per_w // CHUNK
    tile = CHUNK * H
    n_vec = tile // LANES
    UNROLL = 8

    mesh = plsc.VectorSubcoreMesh(core_axis_name="c", subcore_axis_name="s")

    @functools.partial(
        pl.kernel,
        mesh=mesh,
        out_type=jax.ShapeDtypeStruct((B * S * H,), jnp.float32),
        scratch_types=[
            pltpu.VMEM((tile,), jnp.float32),
            pltpu.VMEM((tile,), jnp.float32),
        ],
    )
    def sc_add(x_hbm, p_hbm, o_hbm, pos_v, buf_v):
        wid = lax.axis_index("s") * NC + lax.axis_index("c")
        base_s = wid * pos_per_w

        def add_body(i, _):
            o = i * (LANES * UNROLL)
            for u in range(UNROLL):
                k = o + u * LANES
                buf_v[pl.ds(k, LANES)] = (
                    buf_v[pl.ds(k, LANES)] + pos_v[pl.ds(k, LANES)]
                )
            return 0

        for chunk in range(n_chunks):
            s0 = base_s + chunk * CHUNK
            pltpu.sync_copy(p_hbm.at[pl.ds(s0 * H, tile)], pos_v)
            for b in range(B):
                off = (b * S + s0) * H
                pltpu.sync_copy(x_hbm.at[pl.ds(off, tile)], buf_v)
                lax.fori_loop(0, n_vec // UNROLL, add_body, 0)
                pltpu.sync_copy(buf_v, o_hbm.at[pl.ds(off, tile)])

    return sc_add


def kernel(concat_embeddings, pos_table):
    B, S, H = concat_embeddings.shape
    sc_add = _make_sc_add(B, S, H)
    out = sc_add(concat_embeddings.reshape(-1), pos_table.reshape(-1))
    return out.reshape(B, S, H)


# speedup vs baseline: 1.0948x; 1.0948x over previous
import functools

import jax
import jax.numpy as jnp
from jax import lax
from jax.experimental import pallas as pl
from jax.experimental.pallas import tpu as pltpu
from jax.experimental.pallas import tpu_sc as plsc

NC = 2
NS = 16
NW = NC * NS
LANES = 16
CHUNK = 2
UNROLL = 2
NSLOT = 3


def _make_sc_add(B, S, H):
    pos_per_w = S // NW
    n_chunks = pos_per_w // CHUNK
    tile = CHUNK * H
    n_vec = tile // LANES

    mesh = plsc.VectorSubcoreMesh(core_axis_name="c", subcore_axis_name="s")

    @functools.partial(
        pl.kernel,
        mesh=mesh,
        out_type=jax.ShapeDtypeStruct((B, S * H), jnp.float32),
        scratch_types=(
            [pltpu.VMEM((tile,), jnp.float32)] * NSLOT
            + [pltpu.VMEM((B, tile), jnp.float32)] * NSLOT
            + [pltpu.SemaphoreType.DMA] * (3 * NSLOT)
        ),
    )
    def sc_add(x_hbm, p_hbm, o_hbm, *bufs):
        pos_v = bufs[0:NSLOT]
        buf_v = bufs[NSLOT:2 * NSLOT]
        psem = bufs[2 * NSLOT:3 * NSLOT]
        isem = bufs[3 * NSLOT:4 * NSLOT]
        osem = bufs[4 * NSLOT:5 * NSLOT]
        wid = lax.axis_index("s") * NC + lax.axis_index("c")
        base = wid * pos_per_w * H

        def start_in(t):
            sl = t % NSLOT
            off = base + t * tile
            hp = pltpu.async_copy(
                p_hbm.at[pl.ds(off, tile)], pos_v[sl], psem[sl])
            hx = pltpu.async_copy(
                x_hbm.at[:, pl.ds(off, tile)], buf_v[sl], isem[sl])
            return hp, hx

        def start_out(t):
            sl = t % NSLOT
            off = base + t * tile
            return pltpu.async_copy(
                buf_v[sl], o_hbm.at[:, pl.ds(off, tile)], osem[sl])

        def make_add(sl):
            def add_body(i, _):
                o = i * (LANES * UNROLL)
                for u in range(UNROLL):
                    k = o + u * LANES
                    sli = pl.ds(k, LANES)
                    pv = pos_v[sl][sli]
                    for b in range(B):
                        buf_v[sl][b, sli] = buf_v[sl][b, sli] + pv
                return 0
            return add_body

        ins = {0: start_in(0), 1: start_in(1)}
        outs = {}
        for t in range(n_chunks):
            sl = t % NSLOT
            for h in ins.pop(t):
                h.wait()
            lax.fori_loop(0, n_vec // UNROLL, make_add(sl), 0)
            if t + 2 < n_chunks:
                if t >= 1:
                    outs.pop(t - 1).wait()
                ins[t + 2] = start_in(t + 2)
            outs[t] = start_out(t)
        for t in sorted(outs):
            outs.pop(t).wait()

    return sc_add


def kernel(concat_embeddings, pos_table):
    B, S, H = concat_embeddings.shape
    sc_add = _make_sc_add(B, S, H)
    out = sc_add(concat_embeddings.reshape(B, S * H), pos_table.reshape(-1))
    return out.reshape(B, S, H)


# speedup vs baseline: 4.0925x; 3.7383x over previous
import functools

import jax
import jax.numpy as jnp
from jax import lax
from jax.experimental import pallas as pl
from jax.experimental.pallas import tpu as pltpu
from jax.experimental.pallas import tpu_sc as plsc

NC = 2
NS = 16
NW = NC * NS
LANES = 16
SCHUNK = 8
HCHUNK = 1024
NSLOT = 3


def _make_sc_add(B, S, H):
    pos_per_w = S // NW
    n_sc = pos_per_w // SCHUNK
    n_hc = H // HCHUNK
    n_tiles = n_sc * n_hc
    n_vec = HCHUNK // LANES

    mesh = plsc.VectorSubcoreMesh(core_axis_name="c", subcore_axis_name="s")

    @functools.partial(
        pl.kernel,
        mesh=mesh,
        out_type=jax.ShapeDtypeStruct((B, S, H), jnp.float32),
        scratch_types=(
            [pltpu.VMEM((SCHUNK, HCHUNK), jnp.float32)] * NSLOT
            + [pltpu.VMEM((B, SCHUNK, HCHUNK), jnp.float32)] * NSLOT
            + [pltpu.SemaphoreType.DMA] * (3 * NSLOT)
        ),
        compiler_params=pltpu.CompilerParams(use_tc_tiling_on_sc=True),
    )
    def sc_add(x_hbm, p_hbm, o_hbm, *bufs):
        pos_v = bufs[0:NSLOT]
        buf_v = bufs[NSLOT:2 * NSLOT]
        psem = bufs[2 * NSLOT:3 * NSLOT]
        isem = bufs[3 * NSLOT:4 * NSLOT]
        osem = bufs[4 * NSLOT:5 * NSLOT]
        wid = lax.axis_index("s") * NC + lax.axis_index("c")
        s_base = wid * pos_per_w

        def tile_slices(t):
            c, hi = divmod(t, n_hc)
            s0 = s_base + c * SCHUNK
            return pl.ds(s0, SCHUNK), pl.ds(hi * HCHUNK, HCHUNK)

        def start_in(t):
            sl = t % NSLOT
            ssl, hsl = tile_slices(t)
            hs = [pltpu.async_copy(p_hbm.at[ssl, hsl], pos_v[sl], psem[sl])]
            for b in range(B):
                hs.append(pltpu.async_copy(
                    x_hbm.at[b, ssl, hsl], buf_v[sl].at[b], isem[sl]))
            return hs

        def start_out(t):
            sl = t % NSLOT
            ssl, hsl = tile_slices(t)
            return [pltpu.async_copy(
                buf_v[sl].at[b], o_hbm.at[b, ssl, hsl], osem[sl])
                for b in range(B)]

        def compute(sl):
            @plsc.parallel_loop(0, n_vec)
            def body(j):
                sli = pl.ds(j * LANES, LANES)
                for s in range(SCHUNK):
                    pv = pos_v[sl][s, sli]
                    for b in range(B):
                        buf_v[sl][b, s, sli] = buf_v[sl][b, s, sli] + pv

        ins = {0: start_in(0), 1: start_in(1)}
        outs = {}
        for t in range(n_tiles):
            for h in ins.pop(t):
                h.wait()
            compute(t % NSLOT)
            if t + 2 < n_tiles:
                if t >= 1:
                    for h in outs.pop(t - 1):
                        h.wait()
                ins[t + 2] = start_in(t + 2)
            outs[t] = start_out(t)
        for t in sorted(outs):
            for h in outs.pop(t):
                h.wait()

    return sc_add


def kernel(concat_embeddings, pos_table):
    B, S, H = concat_embeddings.shape
    sc_add = _make_sc_add(B, S, H)
    return sc_add(concat_embeddings, pos_table)


# speedup vs baseline: 4.1255x; 1.0080x over previous
import functools

import jax
import jax.numpy as jnp
from jax import lax
from jax.experimental import pallas as pl
from jax.experimental.pallas import tpu as pltpu
from jax.experimental.pallas import tpu_sc as plsc

NC = 2
NS = 16
NW = NC * NS
LANES = 16
SCHUNK = 8
HCHUNK = 1024
NSLOT = 3


def _make_sc_add(B, S, H):
    pos_per_w = S // NW
    n_sc = pos_per_w // SCHUNK
    n_hc = H // HCHUNK
    n_tiles = n_sc * n_hc
    n_vec = HCHUNK // LANES

    mesh = plsc.VectorSubcoreMesh(core_axis_name="c", subcore_axis_name="s")

    @functools.partial(
        pl.kernel,
        mesh=mesh,
        out_type=jax.ShapeDtypeStruct((B, S, H), jnp.float32),
        scratch_types=(
            [pltpu.VMEM((SCHUNK, HCHUNK), jnp.float32)] * NSLOT
            + [pltpu.VMEM((B, SCHUNK, HCHUNK), jnp.float32)] * NSLOT
            + [pltpu.SemaphoreType.DMA] * (3 * NSLOT)
        ),
        compiler_params=pltpu.CompilerParams(use_tc_tiling_on_sc=True),
    )
    def sc_add(x_hbm, p_hbm, o_hbm, *bufs):
        pos_v = bufs[0:NSLOT]
        buf_v = bufs[NSLOT:2 * NSLOT]
        psem = bufs[2 * NSLOT:3 * NSLOT]
        isem = bufs[3 * NSLOT:4 * NSLOT]
        osem = bufs[4 * NSLOT:5 * NSLOT]
        wid = lax.axis_index("s") * NC + lax.axis_index("c")
        s_base = wid * pos_per_w

        def tile_slices(t):
            c, hi = divmod(t, n_hc)
            s0 = s_base + c * SCHUNK
            return pl.ds(s0, SCHUNK), pl.ds(hi * HCHUNK, HCHUNK)

        def start_in(t):
            sl = t % NSLOT
            ssl, hsl = tile_slices(t)
            return [
                pltpu.async_copy(p_hbm.at[ssl, hsl], pos_v[sl], psem[sl]),
                pltpu.async_copy(x_hbm.at[:, ssl, hsl], buf_v[sl], isem[sl]),
            ]

        def start_out(t):
            sl = t % NSLOT
            ssl, hsl = tile_slices(t)
            return [pltpu.async_copy(
                buf_v[sl], o_hbm.at[:, ssl, hsl], osem[sl])]

        def compute(sl):
            @plsc.parallel_loop(0, n_vec)
            def body(j):
                sli = pl.ds(j * LANES, LANES)
                for s in range(SCHUNK):
                    pv = pos_v[sl][s, sli]
                    for b in range(B):
                        buf_v[sl][b, s, sli] = buf_v[sl][b, s, sli] + pv

        ins = {0: start_in(0), 1: start_in(1)}
        outs = {}
        for t in range(n_tiles):
            for h in ins.pop(t):
                h.wait()
            compute(t % NSLOT)
            if t + 2 < n_tiles:
                if t >= 1:
                    for h in outs.pop(t - 1):
                        h.wait()
                ins[t + 2] = start_in(t + 2)
            outs[t] = start_out(t)
        for t in sorted(outs):
            for h in outs.pop(t):
                h.wait()

    return sc_add


def kernel(concat_embeddings, pos_table):
    B, S, H = concat_embeddings.shape
    sc_add = _make_sc_add(B, S, H)
    return sc_add(concat_embeddings, pos_table)


# speedup vs baseline: 4.1389x; 1.0033x over previous
import functools

import jax
import jax.numpy as jnp
from jax import lax
from jax.experimental import pallas as pl
from jax.experimental.pallas import tpu as pltpu
from jax.experimental.pallas import tpu_sc as plsc

NC = 2
NS = 16
NW = NC * NS
LANES = 16
SCHUNK = 8
HCHUNK = 1024
NSLOT = 3


def _make_sc_add(B, S, H):
    pos_per_w = S // NW
    n_sc = pos_per_w // SCHUNK
    n_hc = H // HCHUNK
    n_tiles = n_sc * n_hc
    n_vec = HCHUNK // LANES

    mesh = plsc.VectorSubcoreMesh(core_axis_name="c", subcore_axis_name="s")

    @functools.partial(
        pl.kernel,
        mesh=mesh,
        out_type=jax.ShapeDtypeStruct((B, S, H), jnp.float32),
        scratch_types=(
            [pltpu.VMEM((SCHUNK, HCHUNK), jnp.float32)] * NSLOT
            + [pltpu.VMEM((B, SCHUNK, HCHUNK), jnp.float32)] * NSLOT
            + [pltpu.SemaphoreType.DMA] * (3 * NSLOT)
        ),
        compiler_params=pltpu.CompilerParams(use_tc_tiling_on_sc=True),
    )
    def sc_add(x_hbm, p_hbm, o_hbm, *bufs):
        pos_v = bufs[0:NSLOT]
        buf_v = bufs[NSLOT:2 * NSLOT]
        psem = bufs[2 * NSLOT:3 * NSLOT]
        isem = bufs[3 * NSLOT:4 * NSLOT]
        osem = bufs[4 * NSLOT:5 * NSLOT]
        wid = lax.axis_index("s") * NC + lax.axis_index("c")
        s_base = wid * pos_per_w

        def tile_slices(t):
            c, hi = divmod(t, n_hc)
            s0 = s_base + c * SCHUNK
            return pl.ds(s0, SCHUNK), pl.ds(hi * HCHUNK, HCHUNK)

        def start_in(t):
            sl = t % NSLOT
            ssl, hsl = tile_slices(t)
            return [
                pltpu.async_copy(p_hbm.at[ssl, hsl], pos_v[sl], psem[sl]),
                pltpu.async_copy(x_hbm.at[:, ssl, hsl], buf_v[sl], isem[sl]),
            ]

        def start_out(t):
            sl = t % NSLOT
            ssl, hsl = tile_slices(t)
            return [pltpu.async_copy(
                buf_v[sl], o_hbm.at[:, ssl, hsl], osem[sl])]

        def compute(sl):
            @plsc.parallel_loop(0, n_vec)
            def body(j):
                sli = pl.ds(j * LANES, LANES)
                for s in range(SCHUNK):
                    pv = pos_v[sl][s, sli]
                    for b in range(B):
                        buf_v[sl][b, s, sli] = buf_v[sl][b, s, sli] + pv

        ins = {0: start_in(0), 1: start_in(1)}
        outs = {}
        for t in range(n_tiles):
            for h in ins.pop(t):
                h.wait()
            compute(t % NSLOT)
            outs[t] = start_out(t)
            if t + 2 < n_tiles:
                if t >= 1:
                    for h in outs.pop(t - 1):
                        h.wait()
                ins[t + 2] = start_in(t + 2)
        for t in sorted(outs):
            for h in outs.pop(t):
                h.wait()

    return sc_add


def kernel(concat_embeddings, pos_table):
    B, S, H = concat_embeddings.shape
    sc_add = _make_sc_add(B, S, H)
    return sc_add(concat_embeddings, pos_table)
